# Initial kernel scaffold; baseline (speedup 1.0000x reference)
import jax
import jax.numpy as jnp
from jax.experimental import pallas as pl


def kernel(points, spoints):
    raise NotImplementedError("write your pallas kernel here")


# speedup vs baseline: 13.1301x; 13.1301x over previous
import jax
import jax.numpy as jnp
from jax.experimental import pallas as pl

_K = 16
_R = 256


def _voroloss_block(sp_ref, pT_ref, out_ref):
    S = sp_ref[...]
    P = pT_ref[...]
    M = S.shape[0]
    R = P.shape[1]

    sx = S[:, 0:1]
    sy = S[:, 1:2]
    sz = S[:, 2:3]
    px = P[0:1, :]
    py = P[1:2, :]
    pz = P[2:3, :]

    f32 = jnp.float32
    s2 = (sx * sx + sy * sy) + sz * sz
    p2 = (px * px + py * py) + pz * pz

    bf = jnp.bfloat16
    sxb = sx.astype(bf).astype(f32)
    syb = sy.astype(bf).astype(f32)
    szb = sz.astype(bf).astype(f32)
    pxb = px.astype(bf).astype(f32)
    pyb = py.astype(bf).astype(f32)
    pzb = pz.astype(bf).astype(f32)
    ps = (sxb * pxb + syb * pyb) + szb * pzb
    d2 = (p2 + s2) - 2.0 * ps

    inf = f32(jnp.inf)
    m0 = jnp.min(d2, axis=0, keepdims=True)
    m = m0
    for _ in range(_K - 1):
        m = jnp.min(jnp.where(d2 > m, d2, inf), axis=0, keepdims=True)
    T = m

    ii = jax.lax.broadcasted_iota(jnp.int32, (M, R), 0)
    i0 = jnp.min(jnp.where(d2 == m0, ii, M), axis=0, keepdims=True)
    sel0 = ii == i0
    zero = f32(0.0)
    s0x = jnp.sum(jnp.where(sel0, sx, zero), axis=0, keepdims=True)
    s0y = jnp.sum(jnp.where(sel0, sy, zero), axis=0, keepdims=True)
    s0z = jnp.sum(jnp.where(sel0, sz, zero), axis=0, keepdims=True)

    g = s2 - 2.0 * ((sx * px + sy * py) + sz * pz)
    g0 = jnp.sum(jnp.where(sel0, g, zero), axis=0, keepdims=True)
    ex = sx - s0x
    ey = sy - s0y
    ez = sz - s0z
    el2 = ex * ex + ey * ey + ez * ez
    diff = g - g0
    f = (diff * diff) / (4.0 * el2)
    mask = (d2 <= T) & jnp.logical_not(sel0)
    res = jnp.min(jnp.where(mask, f, inf), axis=0, keepdims=True)
    out_ref[...] = res[None]


@jax.jit
def kernel(points, spoints):
    N = points.shape[0]
    M = spoints.shape[0]
    R = _R
    grid = N // R
    pT = points.T
    out = pl.pallas_call(
        _voroloss_block,
        grid=(grid,),
        in_specs=[
            pl.BlockSpec((M, 3), lambda i: (0, 0)),
            pl.BlockSpec((3, R), lambda i: (0, i)),
        ],
        out_specs=pl.BlockSpec((1, 1, R), lambda i: (i, 0, 0)),
        out_shape=jax.ShapeDtypeStruct((grid, 1, R), jnp.float32),
    )(spoints, pT)
    return out.reshape(N)
